# Initial kernel scaffold; baseline (speedup 1.0000x reference)
import jax
import jax.numpy as jnp
from jax.experimental import pallas as pl


def kernel(x, indices, output_size, out):
    raise NotImplementedError("write your pallas kernel here")


# speedup vs baseline: 3.8200x; 3.8200x over previous
import dataclasses
import functools

import jax
import jax.numpy as jnp
from jax import lax
from jax.experimental import pallas as pl
from jax.experimental.pallas import tpu as pltpu
from jax.experimental.pallas import tpu_sc as plsc

_LANES = 16
_NUM_WORKERS = 32


@functools.partial(jax.jit, static_argnums=(2,))
def _scatter_sorted(keys2, vals2, s_out):
    p, s_in = keys2.shape
    planes_per_w = p // _NUM_WORKERS
    mesh = plsc.VectorSubcoreMesh(core_axis_name="c", subcore_axis_name="s")
    cp = pltpu.CompilerParams()
    if "needs_layout_passes" in pltpu.CompilerParams.__dataclass_fields__:
        cp = dataclasses.replace(cp, needs_layout_passes=False)

    @functools.partial(
        pl.kernel,
        out_type=jax.ShapeDtypeStruct((p, s_out), jnp.float32),
        mesh=mesh,
        compiler_params=cp,
        scratch_types=[
            pltpu.VMEM((s_in,), jnp.int32),
            pltpu.VMEM((s_in,), jnp.float32),
            pltpu.VMEM((s_out,), jnp.float32),
        ],
    )
    def run(keys_hbm, vals_hbm, out_hbm, idx_v, x_v, plane_v):
        cid = lax.axis_index("c")
        sid = lax.axis_index("s")
        wid = sid * 2 + cid
        zero = jnp.zeros((_LANES,), jnp.float32)

        @pl.loop(0, s_out, step=_LANES)
        def _(i):
            plane_v[pl.ds(i, _LANES)] = zero

        @pl.loop(0, planes_per_w)
        def _(k):
            plane = wid * planes_per_w + k
            base = plane * s_out
            pltpu.sync_copy(keys_hbm.at[plane], idx_v)
            pltpu.sync_copy(vals_hbm.at[plane], x_v)

            @pl.loop(0, s_in, step=_LANES)
            def _(j):
                ii = idx_v[pl.ds(j, _LANES)] - base
                vv = x_v[pl.ds(j, _LANES)]
                plsc.store_scatter(plane_v, [ii], vv)

            pltpu.sync_copy(plane_v, out_hbm.at[plane])

            @pl.loop(0, s_in, step=_LANES)
            def _(j):
                ii = idx_v[pl.ds(j, _LANES)] - base
                plsc.store_scatter(plane_v, [ii], zero)

    return run(keys2, vals2)


def kernel(x, indices, output_size, out):
    n, c, h, w = x.shape
    h_out, w_out = out.shape[2], out.shape[3]
    p = n * c
    s_in = h * w
    s_out = h_out * w_out
    spatial_t = (jnp.asarray(output_size) * jnp.asarray(output_size)).astype(jnp.int32)
    idx = indices.reshape(p, s_in).astype(jnp.int32)
    offsets = (jnp.arange(p, dtype=jnp.int32) * spatial_t).reshape(p, 1)
    flat_idx = (idx + offsets).reshape(-1)
    keys, vals = lax.sort(
        (flat_idx, x.reshape(-1)), dimension=0, is_stable=False, num_keys=1
    )
    res = _scatter_sorted(
        keys.reshape(p, s_in), vals.reshape(p, s_in), s_out
    )
    return res.reshape(n, c, h_out, w_out)
